# baseline (speedup 1.0000x reference)
import jax
import jax.numpy as jnp
import numpy as np
from jax import lax
from jax.experimental import pallas as pl
from jax.experimental.pallas import tpu as pltpu
from jax.experimental.pallas import tpu_sc as plsc

_N = 256
_M = 512
_W = 6
_HSH = _W // 2 - 1
_BETA = 2.30 * (_W / 2.0)
_NBLK = 64
_BSTRIDE = 8
_K = 32768
_NWORK = 32
_PERW = _K // _NWORK
_CHUNK = 64
_NCHUNK = _PERW // _CHUNK
_NGRP = _CHUNK // 16


def _window_np(u):
    t = np.maximum(1.0 - (2.0 * u / _W) ** 2, 0.0)
    return np.where(np.abs(u) <= _W / 2, np.exp(_BETA * (np.sqrt(t) - 1.0)), 0.0)


def _build_constants():
    g = np.arange(_N) - _N // 2
    u = np.linspace(-_W / 2, _W / 2, 4001)
    pu = _window_np(u)
    ft = np.trapezoid(
        pu[None, :] * np.cos(2 * np.pi * (g / _M)[:, None] * u[None, :]), u, axis=1)
    d = 1.0 / ft
    p = np.arange(_M) - _M // 2
    ang = -2.0 * np.pi * np.outer(p, g) / _M
    ar = (np.cos(ang) * d[None, :]).astype(np.float32)
    ai = (np.sin(ang) * d[None, :]).astype(np.float32)
    tt = (np.cos(np.pi * (np.arange(64) + 0.5) / 64) + 1.0) / 2.0
    coef = np.stack(
        [np.polyfit(tt, _window_np((a - _HSH) - tt), 7) for a in range(_W)])
    return ar, ai, coef


_AR, _AI, _COEF = _build_constants()


def _grid_kernel(ar_ref, ai_ref, atr_ref, ati_ref, xr_ref, xi_ref, dcf_ref,
                 t32_ref, sd_ref):
    hi = jax.lax.Precision.HIGHEST
    f32 = jnp.float32
    ar = ar_ref[:, :]
    ai = ai_ref[:, :]
    xr = xr_ref[:, :]
    xi = xi_ref[:, :]
    br = jnp.dot(ar, xr, precision=hi, preferred_element_type=f32) - jnp.dot(
        ai, xi, precision=hi, preferred_element_type=f32)
    bi = jnp.dot(ar, xi, precision=hi, preferred_element_type=f32) + jnp.dot(
        ai, xr, precision=hi, preferred_element_type=f32)
    atr = atr_ref[:, :]
    ati = ati_ref[:, :]
    gr = jnp.dot(br, atr, precision=hi, preferred_element_type=f32) - jnp.dot(
        bi, ati, precision=hi, preferred_element_type=f32)
    gi = jnp.dot(br, ati, precision=hi, preferred_element_type=f32) + jnp.dot(
        bi, atr, precision=hi, preferred_element_type=f32)
    grp = jnp.concatenate([gr, gr[:, :16]], axis=1)
    gip = jnp.concatenate([gi, gi[:, :16]], axis=1)
    for j in range(_NBLK):
        t32_ref[j * _M:(j + 1) * _M, 0:16] = grp[:, j * _BSTRIDE:j * _BSTRIDE + 16]
        t32_ref[j * _M:(j + 1) * _M, 16:32] = gip[:, j * _BSTRIDE:j * _BSTRIDE + 16]
    sd_ref[:, :] = jnp.sqrt(dcf_ref[:, :])


def _horner(coef_row, t):
    w = float(coef_row[0])
    for c in coef_row[1:]:
        w = w * t + float(c)
    return w


def _interp_kernel(t32_hbm, kx_hbm, ky_hbm, sd_hbm, out_hbm,
                   kxv, kyv, sdv, idx0, idx1, gbuf0, gbuf1, outv, sem0, sem1):
    wid = lax.axis_index("s") * 2 + lax.axis_index("c")
    base = wid * _PERW
    pltpu.sync_copy(kx_hbm.at[pl.ds(base, _PERW)], kxv)
    pltpu.sync_copy(ky_hbm.at[pl.ds(base, _PERW)], kyv)
    pltpu.sync_copy(sd_hbm.at[pl.ds(base, _PERW)], sdv)
    lane = lax.iota(jnp.int32, 16)
    lane2 = lane * 2

    def phase_a(c, idxv):
        off = c * _CHUNK
        for gidx in range(_NGRP):
            kxg = kxv[pl.ds(off + gidx * 16, 16)]
            kyg = kyv[pl.ds(off + gidx * 16, 16)]
            fxi = (kxg * float(_M) + float(_M // 2)).astype(jnp.int32)
            fyi = (kyg * float(_M) + float(_M // 2)).astype(jnp.int32)
            qs = (fyi + (512 - _HSH)) & 511
            qb0 = qs >> 3
            for a in range(_W):
                pa = (fxi + (512 - _HSH + a)) & 511
                idxv[pl.ds(a * _CHUNK + gidx * 16, 16)] = qb0 * _M + pa

    def phase_b(idxv, gbuf, sem):
        for i in range(_W):
            pltpu.async_copy(
                t32_hbm.at[idxv.at[pl.ds(i * _CHUNK, _CHUNK)]],
                gbuf.at[pl.ds(i * _CHUNK, _CHUNK)], sem)

    def phase_c(gbuf, sem):
        for i in range(_W):
            pltpu.make_async_copy(
                t32_hbm.at[idx0.at[pl.ds(i * _CHUNK, _CHUNK)]],
                gbuf.at[pl.ds(i * _CHUNK, _CHUNK)], sem).wait()

    def phase_d(c, gbuf):
        off = c * _CHUNK
        for gidx in range(_NGRP):
            kxg = kxv[pl.ds(off + gidx * 16, 16)]
            kyg = kyv[pl.ds(off + gidx * 16, 16)]
            px = kxg * float(_M) + float(_M // 2)
            py = kyg * float(_M) + float(_M // 2)
            fxi = px.astype(jnp.int32)
            fyi = py.astype(jnp.int32)
            fracx = px - fxi.astype(jnp.float32)
            fracy = py - fyi.astype(jnp.float32)
            qs = (fyi + (512 - _HSH)) & 511
            col0 = qs & 7
            wx = [_horner(_COEF[a], fracx) for a in range(_W)]
            wy = [_horner(_COEF[b], fracy) for b in range(_W)]
            rows = [a * _CHUNK + gidx * 16 + lane for a in range(_W)]
            accr = jnp.zeros((16,), jnp.float32)
            acci = jnp.zeros((16,), jnp.float32)
            for b in range(_W):
                colv = col0 + b
                colv1 = colv + 16
                wyb = wy[b]
                for a in range(_W):
                    re = plsc.load_gather(gbuf, [rows[a], colv])
                    im = plsc.load_gather(gbuf, [rows[a], colv1])
                    w = wx[a] * wyb
                    accr = accr + w * re
                    acci = acci + w * im
            sdg = sdv[pl.ds(off + gidx * 16, 16)]
            opos = gidx * 32 + lane2
            plsc.store_scatter(outv, [opos], accr * sdg)
            plsc.store_scatter(outv, [opos + 1], acci * sdg)
        pltpu.sync_copy(outv, out_hbm.at[pl.ds(base * 2 + c * (2 * _CHUNK),
                                               2 * _CHUNK)])

    phase_a(0, idx0)
    phase_b(idx0, gbuf0, sem0)

    def pair_body(c2, carry):
        ce = c2 * 2
        co = ce + 1
        phase_a(co, idx1)
        phase_b(idx1, gbuf1, sem1)
        phase_c(gbuf0, sem0)
        phase_d(ce, gbuf0)

        @pl.when(c2 < _NCHUNK // 2 - 1)
        def _():
            phase_a(ce + 2, idx0)
            phase_b(idx0, gbuf0, sem0)

        phase_c(gbuf1, sem1)
        phase_d(co, gbuf1)
        return carry

    lax.fori_loop(0, _NCHUNK // 2, pair_body, 0)


_interp_call = pl.kernel(
    _interp_kernel,
    out_type=jax.ShapeDtypeStruct((2 * _K,), jnp.float32),
    mesh=plsc.VectorSubcoreMesh(
        core_axis_name="c", subcore_axis_name="s", num_cores=2,
        num_subcores=16),
    scratch_types=[
        pltpu.VMEM((_PERW,), jnp.float32),
        pltpu.VMEM((_PERW,), jnp.float32),
        pltpu.VMEM((_PERW,), jnp.float32),
        pltpu.VMEM((_CHUNK * _W,), jnp.int32),
        pltpu.VMEM((_CHUNK * _W,), jnp.int32),
        pltpu.VMEM((_CHUNK * _W, 32), jnp.float32),
        pltpu.VMEM((_CHUNK * _W, 32), jnp.float32),
        pltpu.VMEM((2 * _CHUNK,), jnp.float32),
        pltpu.SemaphoreType.DMA,
        pltpu.SemaphoreType.DMA,
    ],
    compiler_params=pltpu.CompilerParams(
        needs_layout_passes=False, use_tc_tiling_on_sc=False),
)


@jax.jit
def kernel(x, trajectory, dcf):
    xr = x[..., 0]
    xi = x[..., 1]
    ar = jnp.asarray(_AR)
    ai = jnp.asarray(_AI)
    dcf2 = dcf.reshape(_N, _K // _N)
    t32, sd2 = pl.pallas_call(
        _grid_kernel,
        out_shape=[
            jax.ShapeDtypeStruct((_M * _NBLK, 32), jnp.float32),
            jax.ShapeDtypeStruct((_N, _K // _N), jnp.float32),
        ],
    )(ar, ai, ar.T, ai.T, xr, xi, dcf2)
    y = _interp_call(t32, trajectory[0], trajectory[1], sd2.reshape(-1))
    return y.reshape(_K, 2)


# speedup vs baseline: 1.1326x; 1.1326x over previous
import jax
import jax.numpy as jnp
import numpy as np
from jax import lax
from jax.experimental import pallas as pl
from jax.experimental.pallas import tpu as pltpu
from jax.experimental.pallas import tpu_sc as plsc

_N = 256
_M = 512
_W = 6
_HSH = _W // 2 - 1
_BETA = 2.30 * (_W / 2.0)
_NBLK = 64
_NROW = 3
_K = 32768
_NWORK = 32
_PERW = _K // _NWORK
_CHUNK = 64
_NCHUNK = _PERW // _CHUNK
_NGRP = _CHUNK // 16


def _window_np(u):
    t = np.maximum(1.0 - (2.0 * u / _W) ** 2, 0.0)
    return np.where(np.abs(u) <= _W / 2, np.exp(_BETA * (np.sqrt(t) - 1.0)), 0.0)


def _build_constants():
    g = np.arange(_N) - _N // 2
    u = np.linspace(-_W / 2, _W / 2, 4001)
    pu = _window_np(u)
    ft = np.trapezoid(
        pu[None, :] * np.cos(2 * np.pi * (g / _M)[:, None] * u[None, :]), u, axis=1)
    d = 1.0 / ft
    p = np.arange(_M) - _M // 2
    ang = -2.0 * np.pi * np.outer(p, g) / _M
    ar = (np.cos(ang) * d[None, :]).astype(np.float32)
    ai = (np.sin(ang) * d[None, :]).astype(np.float32)
    tt = (np.cos(np.pi * (np.arange(64) + 0.5) / 64) + 1.0) / 2.0
    coef = np.stack(
        [np.polyfit(tt, _window_np((a - _HSH) - tt), 7) for a in range(_W)])
    perm = 4 * (np.arange(_M) % 128) + np.arange(_M) // 128
    return ar[perm], ai[perm], ar.T.copy(), ai.T.copy(), coef


_AR, _AI, _ATR, _ATI, _COEF = _build_constants()


def _grid_kernel(ar_ref, ai_ref, atr_ref, ati_ref, xr_ref, xi_ref, dcf_ref,
                 t32_ref, sd_ref):
    hi = jax.lax.Precision.HIGHEST
    f32 = jnp.float32
    ar = ar_ref[:, :]
    ai = ai_ref[:, :]
    xr = xr_ref[:, :]
    xi = xi_ref[:, :]
    br = jnp.dot(ar, xr, precision=hi, preferred_element_type=f32) - jnp.dot(
        ai, xi, precision=hi, preferred_element_type=f32)
    bi = jnp.dot(ar, xi, precision=hi, preferred_element_type=f32) + jnp.dot(
        ai, xr, precision=hi, preferred_element_type=f32)
    atr = atr_ref[:, :]
    ati = ati_ref[:, :]
    gr = jnp.dot(br, atr, precision=hi, preferred_element_type=f32) - jnp.dot(
        bi, ati, precision=hi, preferred_element_type=f32)
    gi = jnp.dot(br, ati, precision=hi, preferred_element_type=f32) + jnp.dot(
        bi, atr, precision=hi, preferred_element_type=f32)
    grp = jnp.concatenate([gr, gr[:, :16]], axis=1)
    gip = jnp.concatenate([gi, gi[:, :16]], axis=1)
    for j in range(_NBLK):
        pieces = []
        for sub in range(4):
            pieces.append(grp[sub * 128:(sub + 1) * 128, 8 * j:8 * j + 16])
            pieces.append(gip[sub * 128:(sub + 1) * 128, 8 * j:8 * j + 16])
        t32_ref[j * 128:(j + 1) * 128, :] = jnp.concatenate(pieces, axis=1)
    sd_ref[:, :] = jnp.sqrt(dcf_ref[:, :])


def _horner(coef_row, t):
    w = float(coef_row[0])
    for c in coef_row[1:]:
        w = w * t + float(c)
    return w


def _interp_kernel(t32_hbm, kx_hbm, ky_hbm, sd_hbm, out_hbm,
                   kxv, kyv, sdv, idx0, idx1, gbuf0, gbuf1, outv, sem0, sem1):
    wid = lax.axis_index("s") * 2 + lax.axis_index("c")
    base = wid * _PERW
    pltpu.sync_copy(kx_hbm.at[pl.ds(base, _PERW)], kxv)
    pltpu.sync_copy(ky_hbm.at[pl.ds(base, _PERW)], kyv)
    pltpu.sync_copy(sd_hbm.at[pl.ds(base, _PERW)], sdv)
    lane = lax.iota(jnp.int32, 16)
    lane2 = lane * 2

    def phase_a(c, idxv):
        off = c * _CHUNK
        for gidx in range(_NGRP):
            kxg = kxv[pl.ds(off + gidx * 16, 16)]
            kyg = kyv[pl.ds(off + gidx * 16, 16)]
            fxi = (kxg * float(_M) + float(_M // 2)).astype(jnp.int32)
            fyi = (kyg * float(_M) + float(_M // 2)).astype(jnp.int32)
            qs = (fyi + (512 - _HSH)) & 511
            qb0 = qs >> 3
            sw = (fxi + (512 - _HSH)) & 511
            pgv = sw >> 2
            for r in range(_NROW):
                pg = (pgv + r) & 127
                idxv[pl.ds(r * _CHUNK + gidx * 16, 16)] = qb0 * 128 + pg

    def phase_b(idxv, gbuf, sem):
        for i in range(_NROW):
            pltpu.async_copy(
                t32_hbm.at[idxv.at[pl.ds(i * _CHUNK, _CHUNK)]],
                gbuf.at[pl.ds(i * _CHUNK, _CHUNK)], sem)

    def phase_c(gbuf, sem):
        for i in range(_NROW):
            pltpu.make_async_copy(
                t32_hbm.at[idx0.at[pl.ds(i * _CHUNK, _CHUNK)]],
                gbuf.at[pl.ds(i * _CHUNK, _CHUNK)], sem).wait()

    def phase_d(c, gbuf):
        off = c * _CHUNK
        for gidx in range(_NGRP):
            kxg = kxv[pl.ds(off + gidx * 16, 16)]
            kyg = kyv[pl.ds(off + gidx * 16, 16)]
            px = kxg * float(_M) + float(_M // 2)
            py = kyg * float(_M) + float(_M // 2)
            fxi = px.astype(jnp.int32)
            fyi = py.astype(jnp.int32)
            fracx = px - fxi.astype(jnp.float32)
            fracy = py - fyi.astype(jnp.float32)
            qs = (fyi + (512 - _HSH)) & 511
            col0 = qs & 7
            sw = (fxi + (512 - _HSH)) & 511
            pgv = sw >> 2
            wx = [_horner(_COEF[a], fracx) for a in range(_W)]
            wy = [_horner(_COEF[b], fracy) for b in range(_W)]
            base16 = gidx * 16 + lane
            rows = []
            cbase = []
            for a in range(_W):
                u = sw + a
                ra = (u >> 2) - pgv
                rows.append(ra * _CHUNK + base16)
                cbase.append((u & 3) * 32 + col0)
            accr = jnp.zeros((16,), jnp.float32)
            acci = jnp.zeros((16,), jnp.float32)
            for b in range(_W):
                wyb = wy[b]
                for a in range(_W):
                    colv = cbase[a] + b
                    re = plsc.load_gather(gbuf, [rows[a], colv])
                    im = plsc.load_gather(gbuf, [rows[a], colv + 16])
                    w = wx[a] * wyb
                    accr = accr + w * re
                    acci = acci + w * im
            sdg = sdv[pl.ds(off + gidx * 16, 16)]
            opos = gidx * 32 + lane2
            plsc.store_scatter(outv, [opos], accr * sdg)
            plsc.store_scatter(outv, [opos + 1], acci * sdg)
        pltpu.sync_copy(outv, out_hbm.at[pl.ds(base * 2 + c * (2 * _CHUNK),
                                               2 * _CHUNK)])

    phase_a(0, idx0)
    phase_b(idx0, gbuf0, sem0)

    def pair_body(c2, carry):
        ce = c2 * 2
        co = ce + 1
        phase_a(co, idx1)
        phase_b(idx1, gbuf1, sem1)
        phase_c(gbuf0, sem0)
        phase_d(ce, gbuf0)

        @pl.when(c2 < _NCHUNK // 2 - 1)
        def _():
            phase_a(ce + 2, idx0)
            phase_b(idx0, gbuf0, sem0)

        phase_c(gbuf1, sem1)
        phase_d(co, gbuf1)
        return carry

    lax.fori_loop(0, _NCHUNK // 2, pair_body, 0)


_interp_call = pl.kernel(
    _interp_kernel,
    out_type=jax.ShapeDtypeStruct((2 * _K,), jnp.float32),
    mesh=plsc.VectorSubcoreMesh(
        core_axis_name="c", subcore_axis_name="s", num_cores=2,
        num_subcores=16),
    scratch_types=[
        pltpu.VMEM((_PERW,), jnp.float32),
        pltpu.VMEM((_PERW,), jnp.float32),
        pltpu.VMEM((_PERW,), jnp.float32),
        pltpu.VMEM((_CHUNK * _NROW,), jnp.int32),
        pltpu.VMEM((_CHUNK * _NROW,), jnp.int32),
        pltpu.VMEM((_CHUNK * _NROW, 128), jnp.float32),
        pltpu.VMEM((_CHUNK * _NROW, 128), jnp.float32),
        pltpu.VMEM((2 * _CHUNK,), jnp.float32),
        pltpu.SemaphoreType.DMA,
        pltpu.SemaphoreType.DMA,
    ],
    compiler_params=pltpu.CompilerParams(needs_layout_passes=False),
)


@jax.jit
def kernel(x, trajectory, dcf):
    xr = x[..., 0]
    xi = x[..., 1]
    ar = jnp.asarray(_AR)
    ai = jnp.asarray(_AI)
    atr = jnp.asarray(_ATR)
    ati = jnp.asarray(_ATI)
    dcf2 = dcf.reshape(_N, _K // _N)
    t32, sd2 = pl.pallas_call(
        _grid_kernel,
        out_shape=[
            jax.ShapeDtypeStruct((128 * _NBLK, 128), jnp.float32),
            jax.ShapeDtypeStruct((_N, _K // _N), jnp.float32),
        ],
    )(ar, ai, atr, ati, xr, xi, dcf2)
    y = _interp_call(t32, trajectory[0], trajectory[1], sd2.reshape(-1))
    return y.reshape(_K, 2)
